# baseline (speedup 1.0000x reference)
import functools

import jax
import jax.numpy as jnp
from jax import lax
from jax.experimental import pallas as pl
from jax.experimental.pallas import tpu as pltpu
from jax.experimental.pallas import tpu_sc as plsc

B = 16384
F = 26
NFEAT = 1000000
NC = 2
NS = 16
NW = NC * NS
BPW = B // NW
CHUNK = BPW * F
L = 16
JB = BPW // L
TPAD = NFEAT + (-NFEAT % 1024)
F0 = F // 2


def _make_sc_kernel():
    mesh = plsc.VectorSubcoreMesh(core_axis_name="c", subcore_axis_name="s")

    @functools.partial(
        pl.kernel,
        out_type=jax.ShapeDtypeStruct((B,), jnp.float32),
        mesh=mesh,
        compiler_params=pltpu.CompilerParams(
            needs_layout_passes=False, use_tc_tiling_on_sc=False),
        scratch_types=[
            pltpu.VMEM((F, BPW), jnp.int32),
            pltpu.VMEM((F0 * BPW + L,), jnp.int32),
            pltpu.VMEM(((F - F0) * BPW,), jnp.int32),
            pltpu.VMEM((F0 * BPW + L,), jnp.float32),
            pltpu.VMEM(((F - F0) * BPW,), jnp.float32),
            pltpu.VMEM((F, BPW), jnp.float32),
            pltpu.VMEM((BPW,), jnp.float32),
            pltpu.VMEM_SHARED((TPAD,), jnp.float32),
            pltpu.SemaphoreType.DMA,
            pltpu.SemaphoreType.DMA,
            pltpu.SemaphoreType.DMA,
        ],
    )
    def body(idsT_hbm, valsT_hbm, table_hbm, out_hbm,
             idr_v, idf0_v, idf1_v, g0_v, g1_v, val_v, out_v, tab_sh,
             gsem, vsem, tsem):
        sid = lax.axis_index("s")
        w = sid * NC + lax.axis_index("c")
        cols = pl.ds(w * BPW, BPW)

        vcp = pltpu.async_copy(valsT_hbm.at[:, cols], val_v, vsem)
        seg = TPAD // NS
        tseg = pl.ds(sid * seg, seg)
        tcp = pltpu.async_copy(table_hbm.at[tseg], tab_sh.at[tseg], tsem)
        pltpu.sync_copy(idsT_hbm.at[:, cols], idr_v)

        n0 = F0 * BPW

        def build0(c, carry):
            f = c // JB
            base = L * lax.rem(c, JB)
            idf0_v[pl.ds(c * L, L)] = idr_v[f, pl.ds(base, L)]
            return carry

        def build1(c, carry):
            f = F0 + c // JB
            base = L * lax.rem(c, JB)
            idf1_v[pl.ds(c * L, L)] = idr_v[f, pl.ds(base, L)]
            return carry

        lax.fori_loop(0, n0 // L, build0, 0)
        lax.fori_loop(0, (CHUNK - n0) // L, build1, 0)
        idf0_v[pl.ds(n0, L)] = lax.iota(jnp.int32, L) * 0 + NFEAT
        tcp.wait()
        plsc.subcore_barrier()

        gcp0 = pltpu.async_copy(tab_sh.at[idf0_v], g0_v, gsem)
        gcp1 = pltpu.async_copy(tab_sh.at[idf1_v], g1_v, tsem)
        vcp.wait()
        gcp0.wait()

        bvec = g0_v[pl.ds(n0, L)]

        def fma0(jb, carry):
            base = jb * L
            acc = bvec
            for f in range(F0):
                acc = acc + g0_v[pl.ds(f * BPW + base, L)] * val_v[f, pl.ds(base, L)]
            out_v[pl.ds(base, L)] = acc
            return carry

        lax.fori_loop(0, JB, fma0, 0)
        gcp1.wait()

        def fma1(jb, carry):
            base = jb * L
            acc = out_v[pl.ds(base, L)]
            for f in range(F0, F):
                acc = acc + (g1_v[pl.ds((f - F0) * BPW + base, L)]
                             * val_v[f, pl.ds(base, L)])
            out_v[pl.ds(base, L)] = acc
            return carry

        lax.fori_loop(0, JB, fma1, 0)
        pltpu.sync_copy(out_v, out_hbm.at[pl.ds(w * BPW, BPW)])

    return body


def kernel(id, value, weight, bias):
    table = jnp.pad(weight, ((0, TPAD - NFEAT), (0, 0)))
    table = table.at[NFEAT, 0].set(bias[0])
    return _make_sc_kernel()(id.T, value.T, table.reshape(-1))


# speedup vs baseline: 1.0296x; 1.0296x over previous
import functools

import jax
import jax.numpy as jnp
from jax import lax
from jax.experimental import pallas as pl
from jax.experimental.pallas import tpu as pltpu
from jax.experimental.pallas import tpu_sc as plsc

B = 16384
F = 26
NFEAT = 1000000
NC = 2
NS = 16
NW = NC * NS
BPW = B // NW
CHUNK = BPW * F
L = 16
JB = BPW // L
TPAD = NFEAT + (-NFEAT % 1024)
F0 = F // 2


def _make_sc_kernel():
    mesh = plsc.VectorSubcoreMesh(core_axis_name="c", subcore_axis_name="s")

    @functools.partial(
        pl.kernel,
        out_type=jax.ShapeDtypeStruct((B,), jnp.float32),
        mesh=mesh,
        compiler_params=pltpu.CompilerParams(
            needs_layout_passes=False, use_tc_tiling_on_sc=False),
        scratch_types=[
            pltpu.VMEM((F, BPW), jnp.int32),
            pltpu.VMEM((F0 * BPW + L,), jnp.int32),
            pltpu.VMEM(((F - F0) * BPW,), jnp.int32),
            pltpu.VMEM((F0 * BPW + L,), jnp.float32),
            pltpu.VMEM(((F - F0) * BPW,), jnp.float32),
            pltpu.VMEM((F, BPW), jnp.float32),
            pltpu.VMEM((BPW,), jnp.float32),
            pltpu.VMEM_SHARED((TPAD,), jnp.float32),
            pltpu.SemaphoreType.DMA,
            pltpu.SemaphoreType.DMA,
            pltpu.SemaphoreType.DMA,
        ],
    )
    def body(idsT_hbm, valsT_hbm, table_hbm, out_hbm,
             idr_v, idf0_v, idf1_v, g0_v, g1_v, val_v, out_v, tab_sh,
             gsem, vsem, tsem):
        sid = lax.axis_index("s")
        w = sid * NC + lax.axis_index("c")
        cols = pl.ds(w * BPW, BPW)

        vcp = pltpu.async_copy(valsT_hbm.at[:, cols], val_v, vsem)
        seg = TPAD // NS
        tseg = pl.ds(sid * seg, seg)
        tcp = pltpu.async_copy(table_hbm.at[tseg], tab_sh.at[tseg], tsem)
        pltpu.sync_copy(idsT_hbm.at[:, cols], idr_v)

        n0 = F0 * BPW

        def build0(c, carry):
            f = c // JB
            base = L * lax.rem(c, JB)
            idf0_v[pl.ds(c * L, L)] = idr_v[f, pl.ds(base, L)]
            return carry

        def build1(c, carry):
            f = F0 + c // JB
            base = L * lax.rem(c, JB)
            idf1_v[pl.ds(c * L, L)] = idr_v[f, pl.ds(base, L)]
            return carry

        lax.fori_loop(0, n0 // L, build0, 0)
        lax.fori_loop(0, (CHUNK - n0) // L, build1, 0)
        idf0_v[pl.ds(n0, L)] = lax.iota(jnp.int32, L) * 0 + NFEAT
        tcp.wait()
        plsc.subcore_barrier()

        gcp0 = pltpu.async_copy(tab_sh.at[idf0_v], g0_v, gsem)
        gcp1 = pltpu.async_copy(tab_sh.at[idf1_v], g1_v, tsem)
        vcp.wait()
        gcp0.wait()

        bvec = g0_v[pl.ds(n0, L)]

        def fma0(jb, carry):
            base = jb * L
            acc = bvec
            for f in range(F0):
                acc = acc + g0_v[pl.ds(f * BPW + base, L)] * val_v[f, pl.ds(base, L)]
            out_v[pl.ds(base, L)] = acc
            return carry

        lax.fori_loop(0, JB, fma0, 0)
        gcp1.wait()

        def fma1(jb, carry):
            base = jb * L
            acc = out_v[pl.ds(base, L)]
            for f in range(F0, F):
                acc = acc + (g1_v[pl.ds((f - F0) * BPW + base, L)]
                             * val_v[f, pl.ds(base, L)])
            out_v[pl.ds(base, L)] = acc
            return carry

        lax.fori_loop(0, JB, fma1, 0)
        pltpu.sync_copy(out_v, out_hbm.at[pl.ds(w * BPW, BPW)])

    return body


def kernel(id, value, weight, bias):
    table = jnp.pad(weight, ((0, TPAD - NFEAT), (0, 0))).reshape(-1)
    return _make_sc_kernel()(id.T, value.T, table) + bias
